# baseline (speedup 1.0000x reference)
import functools

import jax
import jax.numpy as jnp
from jax import lax
from jax.experimental import pallas as pl
from jax.experimental.pallas import tpu as pltpu
from jax.experimental.pallas import tpu_sc as plsc

_VOCAB_TILE = 1280


def _sc_gather_rows(table, ids_padded, pad_b, d):
    info = plsc.get_sparse_core_info()
    num_workers = info.num_cores * info.num_subcores
    b_per_w = pad_b // num_workers
    mesh = plsc.VectorSubcoreMesh(core_axis_name="c", subcore_axis_name="s")

    @functools.partial(
        pl.kernel,
        mesh=mesh,
        out_type=jax.ShapeDtypeStruct((pad_b, d), jnp.float32),
        scratch_types=[
            pltpu.VMEM((b_per_w,), jnp.int32),
            pltpu.VMEM((b_per_w, d), jnp.float32),
            pltpu.SemaphoreType.DMA,
        ],
    )
    def gather_kernel(table_hbm, idx_hbm, out_hbm, idx_v, rows_v, sem):
        wid = lax.axis_index("s") * info.num_cores + lax.axis_index("c")
        base = wid * b_per_w
        pltpu.sync_copy(idx_hbm.at[pl.ds(base, b_per_w)], idx_v)
        pltpu.async_copy(table_hbm.at[idx_v], rows_v, sem).wait()
        pltpu.sync_copy(rows_v, out_hbm.at[pl.ds(base, b_per_w)])

    return gather_kernel(table, ids_padded)


def _mm_body(x_ref, w_ref, o_ref):
    x = x_ref[...].astype(jnp.bfloat16)
    w = w_ref[...].astype(jnp.bfloat16)
    o_ref[...] = jnp.dot(x, w, preferred_element_type=jnp.float32)


def _tc_matmul(x, w):
    n, d = x.shape
    _, v = w.shape
    grid = (v // _VOCAB_TILE,)
    return pl.pallas_call(
        _mm_body,
        grid=grid,
        in_specs=[
            pl.BlockSpec((n, d), lambda j: (0, 0)),
            pl.BlockSpec((d, _VOCAB_TILE), lambda j: (0, j)),
        ],
        out_specs=pl.BlockSpec((n, _VOCAB_TILE), lambda j: (0, j)),
        out_shape=jax.ShapeDtypeStruct((n, v), jnp.float32),
    )(x, w)


def kernel(tokenized_prompts, tokenized_text, visual_inputs, embed_table,
           prefix_prompt, mid_prompt, suffix_prompt, lm_head):
    b, t = tokenized_text.shape
    vocab = lm_head.shape[1]
    d = embed_table.shape[1]
    n = b * (t - 1)
    pad_b = ((n + 255) // 256) * 256

    ids = tokenized_text[:, 1:].reshape(-1).astype(jnp.int32)
    ids_padded = jnp.concatenate(
        [ids, jnp.zeros((pad_b - n,), jnp.int32)])
    rows = _sc_gather_rows(embed_table, ids_padded, pad_b, d)
    logits = _tc_matmul(rows[:n], lm_head)
    return logits.reshape(b, t - 1, vocab)


# speedup vs baseline: 1.2317x; 1.2317x over previous
import functools

import jax
import jax.numpy as jnp
from jax import lax
from jax.experimental import pallas as pl
from jax.experimental.pallas import tpu as pltpu
from jax.experimental.pallas import tpu_sc as plsc

_VOCAB_TILE = 1280


def _sc_gather_rows(table, ids_padded, pad_b, d):
    info = plsc.get_sparse_core_info()
    num_workers = info.num_cores * info.num_subcores
    b_per_w = pad_b // num_workers
    mesh = plsc.VectorSubcoreMesh(core_axis_name="c", subcore_axis_name="s")

    @functools.partial(
        pl.kernel,
        mesh=mesh,
        out_type=jax.ShapeDtypeStruct((pad_b, d), jnp.float32),
        scratch_types=[
            pltpu.VMEM((b_per_w,), jnp.int32),
            pltpu.VMEM((b_per_w, d), jnp.float32),
            pltpu.SemaphoreType.DMA,
        ],
    )
    def gather_kernel(table_hbm, idx_hbm, out_hbm, idx_v, rows_v, sem):
        wid = lax.axis_index("s") * info.num_cores + lax.axis_index("c")
        base = wid * b_per_w
        pltpu.sync_copy(idx_hbm.at[pl.ds(base, b_per_w)], idx_v)
        pltpu.async_copy(table_hbm.at[idx_v], rows_v, sem).wait()
        pltpu.sync_copy(rows_v, out_hbm.at[pl.ds(base, b_per_w)])

    return gather_kernel(table, ids_padded)


def _make_mm_body(b, tp):
    def _mm_body(x_ref, w_ref, o_ref):
        x = x_ref[...].astype(jnp.bfloat16)
        w = w_ref[...].astype(jnp.bfloat16)
        res = jnp.dot(x, w, preferred_element_type=jnp.float32)
        for i in range(b):
            o_ref[i, :, :] = res[i * tp:i * tp + tp - 1, :]
    return _mm_body


def _tc_matmul(x, w, b, tp):
    n, d = x.shape
    _, v = w.shape
    grid = (v // _VOCAB_TILE,)
    return pl.pallas_call(
        _make_mm_body(b, tp),
        grid=grid,
        in_specs=[
            pl.BlockSpec((n, d), lambda j: (0, 0)),
            pl.BlockSpec((d, _VOCAB_TILE), lambda j: (0, j)),
        ],
        out_specs=pl.BlockSpec((b, tp - 1, _VOCAB_TILE), lambda j: (0, 0, j)),
        out_shape=jax.ShapeDtypeStruct((b, tp - 1, v), jnp.float32),
    )(x, w)


def kernel(tokenized_prompts, tokenized_text, visual_inputs, embed_table,
           prefix_prompt, mid_prompt, suffix_prompt, lm_head):
    b, t = tokenized_text.shape
    d = embed_table.shape[1]
    pad_b = b * t

    ids = jnp.pad(tokenized_text[:, 1:].astype(jnp.int32),
                  ((0, 0), (0, 1))).reshape(-1)
    rows = _sc_gather_rows(embed_table, ids, pad_b, d)
    return _tc_matmul(rows, lm_head, b, t)


# speedup vs baseline: 1.5178x; 1.2323x over previous
import functools

import jax
import jax.numpy as jnp
from jax import lax
from jax.experimental import pallas as pl
from jax.experimental.pallas import tpu as pltpu
from jax.experimental.pallas import tpu_sc as plsc

_VOCAB_TILE = 1280


def _sc_gather_rows(table, ids_padded, pad_b, d):
    info = plsc.get_sparse_core_info()
    num_workers = info.num_cores * info.num_subcores
    b_per_w = pad_b // num_workers
    mesh = plsc.VectorSubcoreMesh(core_axis_name="c", subcore_axis_name="s")

    @functools.partial(
        pl.kernel,
        mesh=mesh,
        out_type=jax.ShapeDtypeStruct((pad_b, d), jnp.float32),
        scratch_types=[
            pltpu.VMEM((b_per_w,), jnp.int32),
            pltpu.VMEM((b_per_w, d), jnp.float32),
            pltpu.SemaphoreType.DMA,
        ],
    )
    def gather_kernel(table_hbm, idx_hbm, out_hbm, idx_v, rows_v, sem):
        wid = lax.axis_index("s") * info.num_cores + lax.axis_index("c")
        base = wid * b_per_w
        pltpu.sync_copy(idx_hbm.at[pl.ds(base, b_per_w)], idx_v)
        pltpu.async_copy(table_hbm.at[idx_v], rows_v, sem).wait()
        pltpu.sync_copy(rows_v, out_hbm.at[pl.ds(base, b_per_w)])

    return gather_kernel(table, ids_padded)


def _mm_body(x_ref, w_ref, o_ref):
    x = x_ref[...].astype(jnp.bfloat16)
    w = w_ref[...].astype(jnp.bfloat16)
    res = jnp.dot(x, w, preferred_element_type=jnp.float32)
    o_ref[...] = res.reshape(o_ref.shape)


def _tc_matmul(x, w, b, tp):
    n, d = x.shape
    _, v = w.shape
    grid = (v // _VOCAB_TILE,)
    return pl.pallas_call(
        _mm_body,
        grid=grid,
        in_specs=[
            pl.BlockSpec((n, d), lambda j: (0, 0)),
            pl.BlockSpec((d, _VOCAB_TILE), lambda j: (0, j)),
        ],
        out_specs=pl.BlockSpec((b, tp, _VOCAB_TILE), lambda j: (0, 0, j)),
        out_shape=jax.ShapeDtypeStruct((b, tp, v), jnp.float32),
    )(x, w)


def kernel(tokenized_prompts, tokenized_text, visual_inputs, embed_table,
           prefix_prompt, mid_prompt, suffix_prompt, lm_head):
    b, t = tokenized_text.shape
    d = embed_table.shape[1]
    pad_b = b * t

    ids = jnp.pad(tokenized_text[:, 1:].astype(jnp.int32),
                  ((0, 0), (0, 1))).reshape(-1)
    rows = _sc_gather_rows(embed_table, ids, pad_b, d)
    return _tc_matmul(rows, lm_head, b, t)[:, :t - 1, :]
